# Initial kernel scaffold; baseline (speedup 1.0000x reference)
import jax
import jax.numpy as jnp
from jax.experimental import pallas as pl


def kernel(x, weight, expert_bias):
    raise NotImplementedError("write your pallas kernel here")


# speedup vs baseline: 3.3093x; 3.3093x over previous
import functools

import jax
import jax.numpy as jnp
from jax.experimental import pallas as pl
from jax.experimental.pallas import tpu as pltpu

DIM = 4096
N_EXPERTS = 64
TOPK = 8
N_GROUPS = 8
GROUP_SIZE = N_EXPERTS // N_GROUPS
TOPK_GROUPS = 4
ROUTE_SCALE = 2.5

NEG = -1e30


def _gate_kernel(x_ref, w_ref, b_ref, wout_ref, iout_ref):
    bt = x_ref.shape[0]
    logits = jax.lax.dot_general(
        x_ref[...], w_ref[...],
        dimension_numbers=(((1,), (1,)), ((), ())),
        preferred_element_type=jnp.float32,
    )
    scores = jax.nn.sigmoid(logits)
    sb = scores + b_ref[...]

    col = jax.lax.broadcasted_iota(jnp.int32, (bt, N_EXPERTS), 1)
    gid = col // GROUP_SIZE

    gmaxes = []
    gmax_b = jnp.full((bt, N_EXPERTS), NEG)
    for j in range(N_GROUPS):
        mj = jnp.max(jnp.where(gid == j, sb, NEG), axis=1, keepdims=True)
        gmaxes.append(mj)
        gmax_b = jnp.where(gid == j, mj, gmax_b)

    count = jnp.zeros((bt, N_EXPERTS), jnp.int32)
    for j in range(N_GROUPS):
        mj = gmaxes[j]
        count = count + jnp.where(
            (mj > gmax_b) | ((mj == gmax_b) & (j < gid)), 1, 0)
    keep = count < TOPK_GROUPS

    masked = jnp.where(keep, sb, NEG)

    idxs = []
    wts = []
    for _ in range(TOPK):
        v = jnp.max(masked, axis=1, keepdims=True)
        idx = jnp.min(jnp.where(masked == v, col, N_EXPERTS),
                      axis=1, keepdims=True)
        sel = col == idx
        w = jnp.sum(jnp.where(sel, scores, 0.0), axis=1, keepdims=True)
        masked = jnp.where(sel, NEG, masked)
        idxs.append(idx)
        wts.append(w)

    weights = jnp.concatenate(wts, axis=1)
    indices = jnp.concatenate(idxs, axis=1)
    weights = weights / jnp.sum(weights, axis=1, keepdims=True) * ROUTE_SCALE

    wout_ref[...] = weights
    iout_ref[...] = indices


@jax.jit
def kernel(x, weight, expert_bias):
    bsz, seq_len, dim = x.shape
    n_tok = bsz * seq_len
    xf = x.reshape(n_tok, dim)
    bias = expert_bias.reshape(1, N_EXPERTS)

    BT = 512
    grid = (n_tok // BT,)

    weights, indices = pl.pallas_call(
        _gate_kernel,
        grid=grid,
        in_specs=[
            pl.BlockSpec((BT, dim), lambda i: (i, 0)),
            pl.BlockSpec((N_EXPERTS, dim), lambda i: (0, 0)),
            pl.BlockSpec((1, N_EXPERTS), lambda i: (0, 0)),
        ],
        out_specs=[
            pl.BlockSpec((BT, TOPK), lambda i: (i, 0)),
            pl.BlockSpec((BT, TOPK), lambda i: (i, 0)),
        ],
        out_shape=[
            jax.ShapeDtypeStruct((n_tok, TOPK), jnp.float32),
            jax.ShapeDtypeStruct((n_tok, TOPK), jnp.int32),
        ],
    )(xf, weight, bias)

    return weights.astype(x.dtype), indices


# speedup vs baseline: 4.8528x; 1.4664x over previous
import functools

import jax
import jax.numpy as jnp
from jax.experimental import pallas as pl
from jax.experimental.pallas import tpu as pltpu

DIM = 4096
N_EXPERTS = 64
TOPK = 8
N_GROUPS = 8
GROUP_SIZE = N_EXPERTS // N_GROUPS
TOPK_GROUPS = 4
ROUTE_SCALE = 2.5

NEG = -1e30


def _gate_kernel(x_ref, w_ref, b_ref, wout_ref, iout_ref):
    bt = x_ref.shape[0]
    logits = jax.lax.dot_general(
        w_ref[...], x_ref[...],
        dimension_numbers=(((1,), (1,)), ((), ())),
        preferred_element_type=jnp.float32,
    )
    scores = jax.nn.sigmoid(logits)
    sb = scores + b_ref[...]

    rid = jax.lax.broadcasted_iota(jnp.int32, (N_EXPERTS, bt), 0)

    gmax = jnp.concatenate(
        [jnp.max(sb[g * GROUP_SIZE:(g + 1) * GROUP_SIZE, :], axis=0,
                 keepdims=True)
         for g in range(N_GROUPS)], axis=0)

    count = jnp.zeros((N_GROUPS, bt), jnp.int32)
    for j in range(N_GROUPS):
        mj = gmax[j:j + 1, :]
        grow = jnp.concatenate(
            [jnp.full((1, bt), 1 if j < g else 0, jnp.int32)
             for g in range(N_GROUPS)], axis=0)
        count = count + jnp.where(mj > gmax, 1, jnp.where(mj == gmax, grow, 0))
    keep = count < TOPK_GROUPS

    keep_rows = jnp.repeat(keep, GROUP_SIZE, axis=0)
    masked = jnp.where(keep_rows, sb, NEG)

    idxs = []
    wts = []
    for _ in range(TOPK):
        v = jnp.max(masked, axis=0, keepdims=True)
        idx = jnp.min(jnp.where(masked == v, rid, N_EXPERTS),
                      axis=0, keepdims=True)
        sel = rid == idx
        w = jnp.sum(jnp.where(sel, scores, 0.0), axis=0, keepdims=True)
        masked = jnp.where(sel, NEG, masked)
        idxs.append(idx)
        wts.append(w)

    weights = jnp.concatenate(wts, axis=0)
    indices = jnp.concatenate(idxs, axis=0)
    weights = weights / jnp.sum(weights, axis=0, keepdims=True) * ROUTE_SCALE

    wout_ref[...] = weights.T
    iout_ref[...] = indices.T


@jax.jit
def kernel(x, weight, expert_bias):
    bsz, seq_len, dim = x.shape
    n_tok = bsz * seq_len
    xf = x.reshape(n_tok, dim)
    bias = expert_bias.reshape(N_EXPERTS, 1)

    BT = 512
    grid = (n_tok // BT,)

    weights, indices = pl.pallas_call(
        _gate_kernel,
        grid=grid,
        in_specs=[
            pl.BlockSpec((BT, dim), lambda i: (i, 0)),
            pl.BlockSpec((N_EXPERTS, dim), lambda i: (0, 0)),
            pl.BlockSpec((N_EXPERTS, 1), lambda i: (0, 0)),
        ],
        out_specs=[
            pl.BlockSpec((BT, TOPK), lambda i: (i, 0)),
            pl.BlockSpec((BT, TOPK), lambda i: (i, 0)),
        ],
        out_shape=[
            jax.ShapeDtypeStruct((n_tok, TOPK), jnp.float32),
            jax.ShapeDtypeStruct((n_tok, TOPK), jnp.int32),
        ],
    )(xf, weight, bias)

    return weights.astype(x.dtype), indices


# speedup vs baseline: 5.2509x; 1.0820x over previous
import functools

import jax
import jax.numpy as jnp
from jax.experimental import pallas as pl
from jax.experimental.pallas import tpu as pltpu

DIM = 4096
N_EXPERTS = 64
TOPK = 8
N_GROUPS = 8
GROUP_SIZE = N_EXPERTS // N_GROUPS
TOPK_GROUPS = 4
ROUTE_SCALE = 2.5

NEG = -1e30


def _gate_kernel(x_ref, w_ref, b_ref, wout_ref, iout_ref):
    bt = x_ref.shape[0]
    logits = jax.lax.dot_general(
        w_ref[...], x_ref[...],
        dimension_numbers=(((1,), (1,)), ((), ())),
        preferred_element_type=jnp.float32,
    )
    scores = jax.nn.sigmoid(logits)
    sb = scores + b_ref[...]

    rid = jax.lax.broadcasted_iota(jnp.int32, (N_EXPERTS, bt), 0)

    gmax = jnp.concatenate(
        [jnp.max(sb[g * GROUP_SIZE:(g + 1) * GROUP_SIZE, :], axis=0,
                 keepdims=True)
         for g in range(N_GROUPS)], axis=0)

    count = jnp.zeros((N_GROUPS, bt), jnp.int32)
    for j in range(N_GROUPS):
        mj = gmax[j:j + 1, :]
        grow = jnp.concatenate(
            [jnp.full((1, bt), 1 if j < g else 0, jnp.int32)
             for g in range(N_GROUPS)], axis=0)
        count = count + jnp.where(mj > gmax, 1, jnp.where(mj == gmax, grow, 0))
    keep = count < TOPK_GROUPS

    keep_rows = jnp.repeat(keep, GROUP_SIZE, axis=0)
    masked = jnp.where(keep_rows, sb, NEG)

    idxs = []
    wts = []
    for _ in range(TOPK):
        v = jnp.max(masked, axis=0, keepdims=True)
        idx = jnp.min(jnp.where(masked == v, rid, N_EXPERTS),
                      axis=0, keepdims=True)
        sel = rid == idx
        w = jnp.sum(jnp.where(sel, scores, 0.0), axis=0, keepdims=True)
        masked = jnp.where(sel, NEG, masked)
        idxs.append(idx)
        wts.append(w)

    weights = jnp.concatenate(wts, axis=0)
    indices = jnp.concatenate(idxs, axis=0)
    weights = weights / jnp.sum(weights, axis=0, keepdims=True) * ROUTE_SCALE

    wout_ref[...] = weights.T
    iout_ref[...] = indices.T


@jax.jit
def kernel(x, weight, expert_bias):
    bsz, seq_len, dim = x.shape
    n_tok = bsz * seq_len
    xf = x.reshape(n_tok, dim)
    bias = expert_bias.reshape(N_EXPERTS, 1)

    BT = 1024
    grid = (n_tok // BT,)

    weights, indices = pl.pallas_call(
        _gate_kernel,
        grid=grid,
        in_specs=[
            pl.BlockSpec((BT, dim), lambda i: (i, 0)),
            pl.BlockSpec((N_EXPERTS, dim), lambda i: (0, 0)),
            pl.BlockSpec((N_EXPERTS, 1), lambda i: (0, 0)),
        ],
        out_specs=[
            pl.BlockSpec((BT, TOPK), lambda i: (i, 0)),
            pl.BlockSpec((BT, TOPK), lambda i: (i, 0)),
        ],
        out_shape=[
            jax.ShapeDtypeStruct((n_tok, TOPK), jnp.float32),
            jax.ShapeDtypeStruct((n_tok, TOPK), jnp.int32),
        ],
    )(xf, weight, bias)

    return weights.astype(x.dtype), indices
